# baseline (speedup 1.0000x reference)
import functools

import jax
import jax.numpy as jnp
from jax import lax
from jax.experimental import pallas as pl
from jax.experimental.pallas import tpu as pltpu
from jax.experimental.pallas import tpu_sc as plsc

_LANES = 128
_ROWS = 512
_SUB = 8

_NC = 2
_NS = 16
_L = 16


def _tc_body(W, nlag, x_ref, val_ref, idx_ref,
             m1_s, m2_s, m3_s, i1_s, i2_s, i3_s):
    R = val_ref.shape[0]
    nchunk = W // _LANES
    neg_inf = jnp.float32(-jnp.inf)
    zero = jnp.float32(0.0)
    big = jnp.int32(1 << 30)

    lane8 = jax.lax.broadcasted_iota(jnp.int32, (_SUB, _LANES), 1)
    minf = jnp.full((_SUB, _LANES), neg_inf)
    zi = jnp.zeros((_SUB, _LANES), jnp.int32)

    for g in range(R // _SUB):
        rows = pl.ds(g * _SUB, _SUB)
        m1, m2, m3 = minf, minf, minf
        i1, i2, i3 = zi, zi, zi
        xc = x_ref[rows, 0:_LANES]
        last = xc[:, :1]
        for c in range(nchunk):
            if c + 1 < nchunk:
                xn = x_ref[rows, (c + 1) * _LANES:(c + 2) * _LANES]
                first_next = xn[:, :1]
            else:
                xn = None
                first_next = xc[:, -1:]
            xl = jnp.concatenate([last, xc[:, :-1]], axis=1)
            xr = jnp.concatenate([xc[:, 1:], first_next], axis=1)
            sc = jnp.where(xc >= jnp.maximum(xl, xr), xc, zero)
            ivec = lane8 + (c * _LANES)
            b1 = sc > m1
            b2 = sc > m2
            b3 = sc > m3
            m3 = jnp.where(b3, jnp.where(b2, m2, sc), m3)
            i3 = jnp.where(b3, jnp.where(b2, i2, ivec), i3)
            m2 = jnp.where(b2, jnp.where(b1, m1, sc), m2)
            i2 = jnp.where(b2, jnp.where(b1, i1, ivec), i2)
            m1 = jnp.where(b1, sc, m1)
            i1 = jnp.where(b1, ivec, i1)
            last = xc[:, -1:]
            xc = xn
        m1_s[rows, :] = m1
        m2_s[rows, :] = m2
        m3_s[rows, :] = m3
        i1_s[rows, :] = i1
        i2_s[rows, :] = i2
        i3_s[rows, :] = i3

    m1 = m1_s[...]
    m2 = m2_s[...]
    m3 = m3_s[...]
    i1 = i1_s[...]
    i2 = i2_s[...]
    i3 = i3_s[...]
    v1 = jnp.max(m1, axis=1, keepdims=True)
    idx1 = jnp.min(jnp.where(m1 == v1, i1, big), axis=1, keepdims=True)
    m1 = jnp.where(i1 == idx1, neg_inf, m1)

    def next_best(m1, m2, m3):
        v = jnp.max(jnp.maximum(m1, jnp.maximum(m2, m3)), axis=1,
                    keepdims=True)
        cand = jnp.minimum(
            jnp.where(m1 == v, i1, big),
            jnp.minimum(jnp.where(m2 == v, i2, big),
                        jnp.where(m3 == v, i3, big)))
        idx = jnp.min(cand, axis=1, keepdims=True)
        return v, idx, (jnp.where(i1 == idx, neg_inf, m1),
                        jnp.where(i2 == idx, neg_inf, m2),
                        jnp.where(i3 == idx, neg_inf, m3))

    v2, idx2, (m1, m2, m3) = next_best(m1, m2, m3)
    v3, idx3, _ = next_best(m1, m2, m3)

    val_ref[...] = jnp.concatenate([v1, v2, v3], axis=1)
    idx_ref[...] = jnp.concatenate([idx1, idx2, idx3], axis=1) - nlag


def _tc_topk(x2, W, nlag):
    N = x2.shape[0]
    R = _ROWS
    out_shape = [
        jax.ShapeDtypeStruct((N, 3), jnp.float32),
        jax.ShapeDtypeStruct((N, 3), jnp.int32),
    ]
    out_spec = pl.BlockSpec((R, 3), lambda i: (i, 0))
    f32s = functools.partial(pltpu.VMEM, (R, _LANES), jnp.float32)
    i32s = functools.partial(pltpu.VMEM, (R, _LANES), jnp.int32)
    return pl.pallas_call(
        functools.partial(_tc_body, W, nlag),
        grid=(N // R,),
        in_specs=[pl.BlockSpec((R, W), lambda i: (i, 0))],
        out_specs=[out_spec, out_spec],
        out_shape=out_shape,
        scratch_shapes=[f32s(), f32s(), f32s(), i32s(), i32s(), i32s()],
    )(x2)


def _sc_gather_build(N, W, interpret=False):
    nw = _NC * _NS
    rpw = N // nw
    half = rpw // 2
    bpr = W // _LANES
    mesh = plsc.VectorSubcoreMesh(core_axis_name="c", subcore_axis_name="s")

    @functools.partial(
        pl.kernel, mesh=mesh, interpret=interpret,
        compiler_params=pltpu.CompilerParams(needs_layout_passes=False),
        out_type=jax.ShapeDtypeStruct((2 * N,), jnp.float32),
        scratch_types=[
            pltpu.VMEM((rpw,), jnp.int32),
            pltpu.VMEM((half,), jnp.int32),
            pltpu.VMEM((half,), jnp.int32),
            pltpu.VMEM((half, _LANES), jnp.float32),
            pltpu.VMEM((half, _LANES), jnp.float32),
            pltpu.VMEM((2 * rpw,), jnp.float32),
            pltpu.SemaphoreType.DMA,
        ],
    )
    def sc_gather(table_hbm, idx_hbm, out_hbm,
                  idxw_v, ia_v, ib_v, ra_v, rb_v, outw_v, sem):
        wid = lax.axis_index("s") * _NC + lax.axis_index("c")
        base = wid * rpw
        pltpu.sync_copy(idx_hbm.at[pl.ds(base, rpw)], idxw_v)
        lanes = jax.lax.broadcasted_iota(jnp.int32, (_L,), 0)
        for h in range(2):
            r0 = h * half
            for t in range(half // _L):
                iv = idxw_v[pl.ds(r0 + t * _L, _L)]
                p0 = jnp.maximum(iv - 1, 0)
                j0 = p0 >> 7
                g = (base + r0 + t * _L) + lanes
                ia_v[pl.ds(t * _L, _L)] = g * bpr + j0
                ib_v[pl.ds(t * _L, _L)] = g * bpr + jnp.minimum(
                    j0 + 1, bpr - 1)
            copies = []
            for q in range(half // _LANES):
                rows_q = pl.ds(q * _LANES, _LANES)
                copies.append(pltpu.async_copy(
                    table_hbm.at[ia_v.at[rows_q]], ra_v.at[rows_q], sem))
                copies.append(pltpu.async_copy(
                    table_hbm.at[ib_v.at[rows_q]], rb_v.at[rows_q], sem))
            for cp in copies:
                cp.wait()
            for t in range(half // _L):
                iv = idxw_v[pl.ds(r0 + t * _L, _L)]
                p0 = jnp.maximum(iv - 1, 0)
                j0 = p0 >> 7
                o0 = p0 - (j0 << 7)
                p2 = jnp.minimum(iv + 1, W - 1)
                o2 = p2 - (j0 << 7)
                rv = lanes + (t * _L)
                n0 = plsc.load_gather(ra_v, [rv, o0])
                n2a = plsc.load_gather(ra_v, [rv, jnp.minimum(o2, 127)])
                n2b = plsc.load_gather(rb_v, [rv, jnp.maximum(o2 - 128, 0)])
                n2 = jnp.where(o2 >= 128, n2b, n2a)
                outw_v[pl.ds(r0 + t * _L, _L)] = n0
                outw_v[pl.ds(rpw + r0 + t * _L, _L)] = n2
        pltpu.sync_copy(outw_v.at[pl.ds(0, rpw)],
                        out_hbm.at[pl.ds(base, rpw)])
        pltpu.sync_copy(outw_v.at[pl.ds(rpw, rpw)],
                        out_hbm.at[pl.ds(N + base, rpw)])

    return sc_gather


@jax.jit
def kernel(xcorr):
    B, C, H, W = xcorr.shape
    N = B * C * H
    nlag = W // 2
    x2 = xcorr.reshape(N, W)
    vals, idxs = _tc_topk(x2, W, nlag)
    idx1 = idxs[:, 0] + nlag
    table = xcorr.reshape(N * W // _LANES, _LANES)
    nb02 = _sc_gather_build(N, W)(table, idx1).reshape(2, N)
    nb = jnp.stack([nb02[0], vals[:, 0], nb02[1]], axis=1)
    shp = (B, C, H, 3)
    return nb.reshape(shp), vals.reshape(shp), idxs.reshape(shp)


# speedup vs baseline: 2.1488x; 2.1488x over previous
import functools

import jax
import jax.numpy as jnp
from jax.experimental import pallas as pl
from jax.experimental.pallas import tpu as pltpu

_LANES = 128
_ROWS = 512
_SUB = 8


def _body(W, nlag, x_ref, nb_ref, val_ref, idx_ref,
          m1_s, m2_s, m3_s, i1_s, i2_s, i3_s, nl_s, nr_s):
    R = nb_ref.shape[0]
    nchunk = W // _LANES
    neg_inf = jnp.float32(-jnp.inf)
    zero = jnp.float32(0.0)
    big = jnp.int32(1 << 30)

    minf = jnp.full((_SUB, _LANES), neg_inf)
    zi = jnp.zeros((_SUB, _LANES), jnp.int32)
    zf = jnp.zeros((_SUB, _LANES), jnp.float32)

    for g in range(R // _SUB):
        r0 = g * _SUB
        rows = pl.ds(r0, _SUB)
        m1, m2, m3 = minf, minf, minf
        i1, i2, i3 = zi, zi, zi
        nl, nr = zf, zf
        xc = x_ref[rows, 0:_LANES]
        last = xc[:, :1]
        for c in range(nchunk):
            if c + 1 < nchunk:
                xn = x_ref[rows, (c + 1) * _LANES:(c + 2) * _LANES]
                first_next = xn[:, :1]
            else:
                xn = None
                first_next = xc[:, -1:]
            xl = jnp.concatenate([last, xc[:, :-1]], axis=1)
            xr = jnp.concatenate([xc[:, 1:], first_next], axis=1)
            sc = jnp.where(xc >= jnp.maximum(xl, xr), xc, zero)
            cvec = jnp.int32(c)
            b1 = sc > m1
            b2 = sc > m2
            b3 = sc > m3
            m3 = jnp.where(b3, jnp.where(b2, m2, sc), m3)
            i3 = jnp.where(b3, jnp.where(b2, i2, cvec), i3)
            m2 = jnp.where(b2, jnp.where(b1, m1, sc), m2)
            i2 = jnp.where(b2, jnp.where(b1, i1, cvec), i2)
            m1 = jnp.where(b1, sc, m1)
            i1 = jnp.where(b1, cvec, i1)
            nl = jnp.where(b1, xl, nl)
            nr = jnp.where(b1, xr, nr)
            last = xc[:, -1:]
            xc = xn
        m1_s[rows, :] = m1
        m2_s[rows, :] = m2
        m3_s[rows, :] = m3
        i1_s[rows, :] = i1
        i2_s[rows, :] = i2
        i3_s[rows, :] = i3
        nl_s[rows, :] = nl
        nr_s[rows, :] = nr

    m1 = m1_s[...]
    m2 = m2_s[...]
    m3 = m3_s[...]
    laneR = jax.lax.broadcasted_iota(jnp.int32, (R, _LANES), 1)
    i1 = (i1_s[...] << 7) + laneR
    i2 = (i2_s[...] << 7) + laneR
    i3 = (i3_s[...] << 7) + laneR
    v1 = jnp.max(m1, axis=1, keepdims=True)
    idx1 = jnp.min(jnp.where(m1 == v1, i1, big), axis=1, keepdims=True)
    cond1 = i1 == idx1
    n0 = jnp.sum(jnp.where(cond1, nl_s[...], zero), axis=1, keepdims=True)
    n2 = jnp.sum(jnp.where(cond1, nr_s[...], zero), axis=1, keepdims=True)
    m1 = jnp.where(cond1, neg_inf, m1)

    def next_best(m1, m2, m3):
        v = jnp.max(jnp.maximum(m1, jnp.maximum(m2, m3)), axis=1,
                    keepdims=True)
        cand = jnp.minimum(
            jnp.where(m1 == v, i1, big),
            jnp.minimum(jnp.where(m2 == v, i2, big),
                        jnp.where(m3 == v, i3, big)))
        idx = jnp.min(cand, axis=1, keepdims=True)
        return v, idx, (jnp.where(i1 == idx, neg_inf, m1),
                        jnp.where(i2 == idx, neg_inf, m2),
                        jnp.where(i3 == idx, neg_inf, m3))

    v2, idx2, (m1, m2, m3) = next_best(m1, m2, m3)
    v3, idx3, _ = next_best(m1, m2, m3)

    nb_ref[...] = jnp.concatenate([n0, v1, n2], axis=1)
    val_ref[...] = jnp.concatenate([v1, v2, v3], axis=1)
    idx_ref[...] = jnp.concatenate([idx1, idx2, idx3], axis=1) - nlag


@jax.jit
def kernel(xcorr):
    B, C, H, W = xcorr.shape
    N = B * C * H
    nlag = W // 2
    R = _ROWS
    x2 = xcorr.reshape(N, W)
    grid = (N // R,)
    out_shape = [
        jax.ShapeDtypeStruct((N, 3), jnp.float32),
        jax.ShapeDtypeStruct((N, 3), jnp.float32),
        jax.ShapeDtypeStruct((N, 3), jnp.int32),
    ]
    out_spec = pl.BlockSpec((R, 3), lambda i: (i, 0))
    f32s = functools.partial(pltpu.VMEM, (R, _LANES), jnp.float32)
    i32s = functools.partial(pltpu.VMEM, (R, _LANES), jnp.int32)
    nb, vals, idxs = pl.pallas_call(
        functools.partial(_body, W, nlag),
        grid=grid,
        in_specs=[pl.BlockSpec((R, W), lambda i: (i, 0))],
        out_specs=[out_spec, out_spec, out_spec],
        out_shape=out_shape,
        scratch_shapes=[f32s(), f32s(), f32s(), i32s(), i32s(), i32s(),
                        f32s(), f32s()],
    )(x2)
    shp = (B, C, H, 3)
    return nb.reshape(shp), vals.reshape(shp), idxs.reshape(shp)
